# baseline (speedup 1.0000x reference)
import functools

import jax
import jax.numpy as jnp
from jax import lax
from jax.experimental import pallas as pl
from jax.experimental.pallas import tpu as pltpu
from jax.experimental.pallas import tpu_sc as plsc

_BATCH = 4
_SEQ = 8192
_B = _BATCH * _SEQ
_D = 64
_NC, _NS = 2, 16
_NW = _NC * _NS
_BPW = _B // _NW
_WPB = _SEQ // _BPW

_CH = 128
_NCH = _BPW // _CH
_NBUF = 4
_DEPTH = 2

_mesh = plsc.VectorSubcoreMesh(core_axis_name="c", subcore_axis_name="s")


@functools.partial(
    pl.kernel,
    mesh=_mesh,
    out_type=jax.ShapeDtypeStruct((_BATCH, _SEQ, 2 * _D), jnp.float32),
    scratch_types=[
        pltpu.VMEM((_BPW,), jnp.int32),
        [pltpu.VMEM((_CH, 2 * _D), jnp.float32) for _ in range(_NBUF)],
        pltpu.SemaphoreType.DMA((_NBUF,)),
        pltpu.SemaphoreType.DMA((_NBUF,)),
    ],
)
def _gather_rows(pos_hbm, ctab_hbm, out_hbm, idx_v, bufs, g_sem, w_sem):
    wid = lax.axis_index("s") * _NC + lax.axis_index("c")
    row = wid // _WPB
    off = (wid % _WPB) * _BPW
    pltpu.sync_copy(pos_hbm.at[row, pl.ds(off, _BPW)], idx_v)

    def start_gather(t):
        idx_sl = idx_v.at[pl.ds(t * _CH, _CH)]
        return pltpu.async_copy(ctab_hbm.at[idx_sl],
                                bufs[t % _NBUF], g_sem.at[t % _NBUF])

    def start_wb(t):
        dst = out_hbm.at[row, pl.ds(off + t * _CH, _CH)]
        return pltpu.async_copy(bufs[t % _NBUF], dst, w_sem.at[t % _NBUF])

    gathers = {t: start_gather(t) for t in range(_DEPTH)}
    wbs = {}
    for w in range(_NCH):
        nx = w + _DEPTH
        if nx < _NCH:
            if nx >= _NBUF:
                wbs[nx - _NBUF].wait()
            gathers[nx] = start_gather(nx)
        gathers[w].wait()
        wbs[w] = start_wb(w)
    for t in range(max(0, _NCH - _NBUF), _NCH):
        wbs[t].wait()


def kernel(position, sin_values, cos_values):
    ctab = jnp.concatenate([sin_values, cos_values], axis=1)
    combined = _gather_rows(position, ctab)
    return combined[..., :_D], combined[..., _D:]
